# baseline (speedup 1.0000x reference)
import functools

import jax
import jax.numpy as jnp
from jax import lax
from jax.experimental import pallas as pl
from jax.experimental.pallas import tpu as pltpu
from jax.experimental.pallas import tpu_sc as plsc

VOCAB = 1000
D = 1000
DM = 896
DT = 128
N_ROWS = 1024 * 200

_info = plsc.get_sparse_core_info()
NC, NS = _info.num_cores, _info.num_subcores
NW = NC * NS
K_PARTS = 4
P_ROWS = N_ROWS // K_PARTS
ROWS_PER_W = P_ROWS // NW
CH = 40
N_CHUNKS = ROWS_PER_W // CH
T = N_CHUNKS // 2


@functools.partial(
    pl.kernel,
    mesh=plsc.VectorSubcoreMesh(core_axis_name="c", subcore_axis_name="s"),
    out_type=(
        jax.ShapeDtypeStruct((P_ROWS, D), jnp.float32),
        jax.ShapeDtypeStruct((P_ROWS, DT), jnp.float32),
    ),
    scratch_types=[
        pltpu.VMEM((CH,), jnp.int32),
        pltpu.VMEM((CH,), jnp.int32),
        pltpu.VMEM((CH, DM), jnp.float32),
        pltpu.VMEM((CH, DM), jnp.float32),
        pltpu.VMEM((CH, DT), jnp.float32),
        pltpu.VMEM((CH, DT), jnp.float32),
        pltpu.SemaphoreType.DMA,
        pltpu.SemaphoreType.DMA,
        pltpu.SemaphoreType.DMA,
        pltpu.SemaphoreType.DMA,
        pltpu.SemaphoreType.DMA,
        pltpu.SemaphoreType.DMA,
    ],
)
def _gather_rows(idx_hbm, tmain_hbm, ttail_hbm, out_hbm, tail_hbm,
                 ib0, ib1, mb0, mb1, tb0, tb1, is0, is1, gs0, gs1, os0, os1):
    wid = lax.axis_index("s") * NC + lax.axis_index("c")
    base = wid * ROWS_PER_W
    ib = (ib0, ib1)
    mb = (mb0, mb1)
    tb = (tb0, tb1)
    isem = (is0, is1)
    gsem = (gs0, gs1)
    osem = (os0, os1)

    def idx_start(b, g):
        pltpu.async_copy(idx_hbm.at[pl.ds(base + g * CH, CH)], ib[b], isem[b])

    def idx_wait(b):
        pltpu.make_async_copy(idx_hbm.at[pl.ds(0, CH)], ib[b], isem[b]).wait()

    def gather_start(b):
        pltpu.async_copy(tmain_hbm.at[ib[b]], mb[b], gsem[b])
        pltpu.async_copy(ttail_hbm.at[ib[b]], tb[b], gsem[b])

    def gather_wait(b):
        pltpu.make_async_copy(tmain_hbm.at[ib[b]], mb[b], gsem[b]).wait()
        pltpu.make_async_copy(ttail_hbm.at[ib[b]], tb[b], gsem[b]).wait()

    def out_start(b, g):
        r = base + g * CH
        pltpu.async_copy(mb[b], out_hbm.at[pl.ds(r, CH), pl.ds(0, DM)], osem[b])
        pltpu.async_copy(tb[b], tail_hbm.at[pl.ds(r, CH)], osem[b])

    def out_wait(b):
        pltpu.make_async_copy(mb[b], out_hbm.at[pl.ds(0, CH), pl.ds(0, DM)],
                              osem[b]).wait()
        pltpu.make_async_copy(tb[b], tail_hbm.at[pl.ds(0, CH)], osem[b]).wait()

    idx_start(0, 0)
    idx_wait(0)
    gather_start(0)
    idx_start(1, 1)

    def body(t, carry):
        a = 2 * t

        gather_wait(0)
        idx_wait(1)

        @pl.when(t >= 1)
        def _():
            out_wait(1)

        gather_start(1)
        out_start(0, a)

        @pl.when(t < T - 1)
        def _():
            idx_start(0, a + 2)

        gather_wait(1)

        @pl.when(t < T - 1)
        def _():
            idx_wait(0)
            out_wait(0)
            gather_start(0)
            idx_start(1, a + 3)

        out_start(1, a + 1)
        return carry

    lax.fori_loop(0, T, body, 0)

    out_wait(0)
    out_wait(1)


def kernel(idx, targets, table):
    flat = idx.reshape(N_ROWS)
    table_main = table[:, :DM]
    table_tail = jnp.pad(table[:, DM:], ((0, 0), (0, DT - (D - DM))))
    B_P = idx.shape[0] // K_PARTS
    parts = []
    for k in range(K_PARTS):
        out, tail = _gather_rows(
            lax.dynamic_slice(flat, (k * P_ROWS,), (P_ROWS,)),
            table_main, table_tail)
        out = lax.dynamic_update_slice(out, tail[:, : D - DM], (0, DM))
        parts.append(out.reshape(B_P, idx.shape[1], D))
    return jnp.concatenate(parts, axis=0)


# speedup vs baseline: 1.5519x; 1.5519x over previous
import functools

import jax
import jax.numpy as jnp
from jax import lax
from jax.experimental import pallas as pl
from jax.experimental.pallas import tpu as pltpu
from jax.experimental.pallas import tpu_sc as plsc

VOCAB = 1000
D = 1000
DM = 896
DT = 128
N_ROWS = 1024 * 200

_info = plsc.get_sparse_core_info()
NC, NS = _info.num_cores, _info.num_subcores
NW = NC * NS
K_PARTS = 4
P_ROWS = N_ROWS // K_PARTS
ROWS_PER_W = P_ROWS // NW
CH = 40
N_CHUNKS = ROWS_PER_W // CH
T = N_CHUNKS // 2


@functools.partial(
    pl.kernel,
    mesh=plsc.VectorSubcoreMesh(core_axis_name="c", subcore_axis_name="s"),
    out_type=(
        jax.ShapeDtypeStruct((P_ROWS, DM), jnp.float32),
        jax.ShapeDtypeStruct((P_ROWS, DT), jnp.float32),
    ),
    scratch_types=[
        pltpu.VMEM((CH,), jnp.int32),
        pltpu.VMEM((CH,), jnp.int32),
        pltpu.VMEM((CH, DM), jnp.float32),
        pltpu.VMEM((CH, DM), jnp.float32),
        pltpu.VMEM((CH, DT), jnp.float32),
        pltpu.VMEM((CH, DT), jnp.float32),
        pltpu.SemaphoreType.DMA,
        pltpu.SemaphoreType.DMA,
        pltpu.SemaphoreType.DMA,
        pltpu.SemaphoreType.DMA,
        pltpu.SemaphoreType.DMA,
        pltpu.SemaphoreType.DMA,
    ],
)
def _gather_rows(idx_hbm, tmain_hbm, ttail_hbm, out_hbm, tail_hbm,
                 ib0, ib1, mb0, mb1, tb0, tb1, is0, is1, gs0, gs1, os0, os1):
    wid = lax.axis_index("s") * NC + lax.axis_index("c")
    base = wid * ROWS_PER_W
    ib = (ib0, ib1)
    mb = (mb0, mb1)
    tb = (tb0, tb1)
    isem = (is0, is1)
    gsem = (gs0, gs1)
    osem = (os0, os1)

    def idx_start(b, g):
        pltpu.async_copy(idx_hbm.at[pl.ds(base + g * CH, CH)], ib[b], isem[b])

    def idx_wait(b):
        pltpu.make_async_copy(idx_hbm.at[pl.ds(0, CH)], ib[b], isem[b]).wait()

    def gather_start(b):
        pltpu.async_copy(tmain_hbm.at[ib[b]], mb[b], gsem[b])
        pltpu.async_copy(ttail_hbm.at[ib[b]], tb[b], gsem[b])

    def gather_wait(b):
        pltpu.make_async_copy(tmain_hbm.at[ib[b]], mb[b], gsem[b]).wait()
        pltpu.make_async_copy(ttail_hbm.at[ib[b]], tb[b], gsem[b]).wait()

    def out_start(b, g):
        r = base + g * CH
        pltpu.async_copy(mb[b], out_hbm.at[pl.ds(r, CH)], osem[b])
        pltpu.async_copy(tb[b], tail_hbm.at[pl.ds(r, CH)], osem[b])

    def out_wait(b):
        pltpu.make_async_copy(mb[b], out_hbm.at[pl.ds(0, CH)], osem[b]).wait()
        pltpu.make_async_copy(tb[b], tail_hbm.at[pl.ds(0, CH)], osem[b]).wait()

    idx_start(0, 0)
    idx_wait(0)
    gather_start(0)
    idx_start(1, 1)

    def body(t, carry):
        a = 2 * t

        gather_wait(0)
        idx_wait(1)

        @pl.when(t >= 1)
        def _():
            out_wait(1)

        gather_start(1)
        out_start(0, a)

        @pl.when(t < T - 1)
        def _():
            idx_start(0, a + 2)

        gather_wait(1)

        @pl.when(t < T - 1)
        def _():
            idx_wait(0)
            out_wait(0)
            gather_start(0)
            idx_start(1, a + 3)

        out_start(1, a + 1)
        return carry

    lax.fori_loop(0, T, body, 0)

    out_wait(0)
    out_wait(1)


B = 1024
L = 200
B_P = B // K_PARTS


def _transpose_part(k):
    LB = 8

    def body(*refs):
        main_ref, tail_ref, out_ref = refs[-3:]
        for i in range(LB):
            out_ref[i, :DM, :] = main_ref[:, i, :].T
            out_ref[i, DM:D, :] = tail_ref[:, i, : D - DM].T

    prev_specs = [] if k == 0 else [pl.BlockSpec(memory_space=pl.ANY)]
    return pl.pallas_call(
        body,
        grid=(L // LB,),
        in_specs=prev_specs + [
            pl.BlockSpec((B_P, LB, DM), lambda l: (0, l, 0)),
            pl.BlockSpec((B_P, LB, DT), lambda l: (0, l, 0)),
        ],
        out_specs=pl.BlockSpec((LB, D, B_P), lambda l: (l, 0, k)),
        out_shape=jax.ShapeDtypeStruct((L, D, B), jnp.float32),
        input_output_aliases={} if k == 0 else {0: 0},
        compiler_params=pltpu.CompilerParams(
            dimension_semantics=("arbitrary",)),
    )


def kernel(idx, targets, table):
    flat = idx.reshape(N_ROWS)
    table_main = table[:, :DM]
    table_tail = jnp.pad(table[:, DM:], ((0, 0), (0, DT - (D - DM))))
    out3 = None
    for k in range(K_PARTS):
        main, tail = _gather_rows(
            lax.dynamic_slice(flat, (k * P_ROWS,), (P_ROWS,)),
            table_main, table_tail)
        main3 = main.reshape(B_P, L, DM)
        tail3 = tail.reshape(B_P, L, DT)
        if out3 is None:
            out3 = _transpose_part(0)(main3, tail3)
        else:
            out3 = _transpose_part(k)(out3, main3, tail3)
    return jnp.transpose(out3, (2, 0, 1))


# speedup vs baseline: 1.5736x; 1.0140x over previous
import functools

import jax
import jax.numpy as jnp
from jax import lax
from jax.experimental import pallas as pl
from jax.experimental.pallas import tpu as pltpu
from jax.experimental.pallas import tpu_sc as plsc

VOCAB = 1000
D = 1000
DM = 896
DT = 128
N_ROWS = 1024 * 200
B = 1024
L = 200

_info = plsc.get_sparse_core_info()
NC, NS = _info.num_cores, _info.num_subcores
NW = NC * NS
K_PARTS = 4
P_ROWS = N_ROWS // K_PARTS
B_P = B // K_PARTS
ROWS_PER_W = P_ROWS // NW
CH = 40
N_CHUNKS = ROWS_PER_W // CH
T = N_CHUNKS // 2


@functools.partial(
    pl.kernel,
    mesh=plsc.VectorSubcoreMesh(core_axis_name="c", subcore_axis_name="s"),
    out_type=jax.ShapeDtypeStruct((P_ROWS, DM), jnp.float32),
    scratch_types=[
        pltpu.VMEM((CH,), jnp.int32),
        pltpu.VMEM((CH,), jnp.int32),
        pltpu.VMEM((CH, DM), jnp.float32),
        pltpu.VMEM((CH, DM), jnp.float32),
        pltpu.SemaphoreType.DMA,
        pltpu.SemaphoreType.DMA,
        pltpu.SemaphoreType.DMA,
        pltpu.SemaphoreType.DMA,
        pltpu.SemaphoreType.DMA,
        pltpu.SemaphoreType.DMA,
    ],
)
def _gather_rows(idx_hbm, tmain_hbm, out_hbm,
                 ib0, ib1, mb0, mb1, is0, is1, gs0, gs1, os0, os1):
    wid = lax.axis_index("s") * NC + lax.axis_index("c")
    base = wid * ROWS_PER_W
    ib = (ib0, ib1)
    mb = (mb0, mb1)
    isem = (is0, is1)
    gsem = (gs0, gs1)
    osem = (os0, os1)

    def idx_start(b, g):
        pltpu.async_copy(idx_hbm.at[pl.ds(base + g * CH, CH)], ib[b], isem[b])

    def idx_wait(b):
        pltpu.make_async_copy(idx_hbm.at[pl.ds(0, CH)], ib[b], isem[b]).wait()

    def gather_start(b):
        pltpu.async_copy(tmain_hbm.at[ib[b]], mb[b], gsem[b])

    def gather_wait(b):
        pltpu.make_async_copy(tmain_hbm.at[ib[b]], mb[b], gsem[b]).wait()

    def out_start(b, g):
        pltpu.async_copy(mb[b], out_hbm.at[pl.ds(base + g * CH, CH)], osem[b])

    def out_wait(b):
        pltpu.make_async_copy(mb[b], out_hbm.at[pl.ds(0, CH)], osem[b]).wait()

    idx_start(0, 0)
    idx_wait(0)
    gather_start(0)
    idx_start(1, 1)

    def body(t, carry):
        a = 2 * t

        gather_wait(0)
        idx_wait(1)

        @pl.when(t >= 1)
        def _():
            out_wait(1)

        gather_start(1)
        out_start(0, a)

        @pl.when(t < T - 1)
        def _():
            idx_start(0, a + 2)

        gather_wait(1)

        @pl.when(t < T - 1)
        def _():
            idx_wait(0)
            out_wait(0)
            gather_start(0)
            idx_start(1, a + 3)

        out_start(1, a + 1)
        return carry

    lax.fori_loop(0, T, body, 0)

    out_wait(0)
    out_wait(1)


def _transpose_part(k):
    LB = 8

    def body(*refs):
        main_ref, idx_ref, tailt_ref, out_ref = refs[-4:]
        for i in range(LB):
            out_ref[i, :DM, :] = main_ref[:, i, :].T
            row = idx_ref[i, :]
            oh = (lax.broadcasted_iota(jnp.int32, (VOCAB, B_P), 0)
                  == row[None, :]).astype(jnp.float32)
            res = lax.dot_general(
                tailt_ref[...], oh, (((1,), (0,)), ((), ())),
                preferred_element_type=jnp.float32,
                precision=lax.Precision.HIGHEST)
            out_ref[i, DM:D, :] = res[: D - DM, :]

    prev_specs = [] if k == 0 else [pl.BlockSpec(memory_space=pl.ANY)]
    return pl.pallas_call(
        body,
        grid=(L // LB,),
        in_specs=prev_specs + [
            pl.BlockSpec((B_P, LB, DM), lambda l: (0, l, 0)),
            pl.BlockSpec((LB, B_P), lambda l: (l, 0)),
            pl.BlockSpec((DT, VOCAB), lambda l: (0, 0)),
        ],
        out_specs=pl.BlockSpec((LB, D, B_P), lambda l: (l, 0, k)),
        out_shape=jax.ShapeDtypeStruct((L, D, B), jnp.float32),
        input_output_aliases={} if k == 0 else {0: 0},
        compiler_params=pltpu.CompilerParams(
            dimension_semantics=("arbitrary",)),
    )


def kernel(idx, targets, table):
    flat = idx.reshape(N_ROWS)
    table_main = table[:, :DM]
    tail_t = jnp.pad(table[:, DM:].T, ((0, DT - (D - DM)), (0, 0)))
    out3 = None
    for k in range(K_PARTS):
        main = _gather_rows(
            lax.dynamic_slice(flat, (k * P_ROWS,), (P_ROWS,)), table_main)
        main3 = main.reshape(B_P, L, DM)
        idx_t = idx[k * B_P:(k + 1) * B_P, :].T
        if out3 is None:
            out3 = _transpose_part(0)(main3, idx_t, tail_t)
        else:
            out3 = _transpose_part(k)(out3, main3, idx_t, tail_t)
    return jnp.transpose(out3, (2, 0, 1))
